# baseline (speedup 1.0000x reference)
import functools
import jax
import jax.numpy as jnp
from jax import lax
from jax.experimental import pallas as pl
from jax.experimental.pallas import tpu as pltpu
from jax.experimental.pallas import tpu_sc as plsc

N = 200000
NP = 200704
CHUNK = NP // 16
NBLK = NP // 1024
MAXP = 35
MAXV = 20000
GX, GY, GZ = 1408, 1600, 40
GXY = GX * GY
SENT = GXY * GZ
K1 = 16384
SHIFT2 = 14
K2 = 5504
VOXROWS = MAXV * MAXP
R4 = 704512
RC = 24576
DMASK = 4095

_mesh = plsc.VectorSubcoreMesh(
    core_axis_name="c", subcore_axis_name="s", num_cores=1, num_subcores=16
)


@functools.partial(
    pl.kernel,
    mesh=_mesh,
    compiler_params=pltpu.CompilerParams(needs_layout_passes=False),
    out_type=[
        jax.ShapeDtypeStruct((R4 * 4,), jnp.float32),
        jax.ShapeDtypeStruct((RC,), jnp.int32),
        jax.ShapeDtypeStruct((RC * 3,), jnp.int32),
    ],
    scratch_types=[
        pltpu.VMEM_SHARED((NP + 16,), jnp.int32),
        pltpu.VMEM_SHARED((NP,), jnp.int32),
        pltpu.VMEM_SHARED((NP,), jnp.int32),
        pltpu.VMEM_SHARED((NP,), jnp.int32),
        pltpu.VMEM((CHUNK,), jnp.float32),
        pltpu.VMEM((CHUNK,), jnp.float32),
        pltpu.VMEM((CHUNK,), jnp.float32),
        pltpu.VMEM((CHUNK,), jnp.int32),
        pltpu.VMEM((K1,), jnp.int32),
        pltpu.VMEM((1040,), jnp.int32),
        pltpu.VMEM((1024,), jnp.int32),
        pltpu.VMEM((16,), jnp.int32),
        pltpu.VMEM((16,), jnp.int32),
        pltpu.VMEM((128,), jnp.int32),
        pltpu.VMEM((128,), jnp.int32),
        pltpu.VMEM((128,), jnp.int32),
        pltpu.VMEM((128,), jnp.int32),
        pltpu.VMEM((128,), jnp.int32),
        pltpu.VMEM((128,), jnp.int32),
        pltpu.VMEM((128,), jnp.int32),
        pltpu.VMEM((128,), jnp.float32),
        pltpu.VMEM((128,), jnp.int32),
        pltpu.VMEM((128,), jnp.int32),
        pltpu.VMEM((128,), jnp.int32),
        pltpu.VMEM((4096,), jnp.float32),
        pltpu.VMEM((4096,), jnp.int32),
    ],
)
def _voxelize_sc(pfl_hbm, zf_hbm, zi_hbm,
                 vox, counts, coors,
                 keya, key1, idx1, idxb,
                 pxb, pyb, pzb, keyb, hist, blkk, blki, tmp16, s16,
                 destr, sidxr, stg1, stg2, crdb, gidxb, dstb, bcol,
                 cz, cy, cx, zfb, zib):
    w = lax.axis_index("s")
    iota = lax.iota(jnp.int32, 16)

    dn = lax.GatherDimensionNumbers(
        offset_dims=(), collapsed_slice_dims=(0,), start_index_map=(0,))

    def take16(x, i):
        return lax.gather(x, i[:, None], dn, (1,),
                          mode=lax.GatherScatterMode.PROMISE_IN_BOUNDS)

    def rank_dedup(d):
        ks2, _ = plsc.sort_key_val(d * 16 + iota, iota)
        dsd = ks2 >> 4
        perm = ks2 & 15
        prev = take16(dsd, jnp.maximum(iota - 1, 0))
        firstm = (iota == 0) | (dsd != prev)
        starts = plsc.cummax(jnp.where(firstm, iota, 0))
        rank_s = iota - starts
        nxt = take16(dsd, jnp.minimum(iota + 1, 15))
        lastm = (iota == 15) | (dsd != nxt)
        plsc.store_scatter(tmp16, [perm], rank_s)
        return tmp16[...], dsd, rank_s + 1, lastm

    pltpu.sync_copy(zf_hbm, zfb)
    pltpu.sync_copy(zi_hbm, zib)

    def zvox(i, _):
        pltpu.sync_copy(zfb, vox.at[pl.ds((w * 43 + i) * 4096, 4096)])
        return 0
    lax.fori_loop(0, 43, zvox, 0)

    pltpu.sync_copy(zib.at[pl.ds(0, 1536)], counts.at[pl.ds(w * 1536, 1536)])
    pltpu.sync_copy(zib, coors.at[pl.ds(w * 4608, 4096)])
    pltpu.sync_copy(zib.at[pl.ds(0, 512)],
                    coors.at[pl.ds(w * 4608 + 4096, 512)])

    base = w * CHUNK
    pltpu.sync_copy(pfl_hbm.at[pl.ds(base, CHUNK)], pxb)
    pltpu.sync_copy(pfl_hbm.at[pl.ds(NP + base, CHUNK)], pyb)
    pltpu.sync_copy(pfl_hbm.at[pl.ds(2 * NP + base, CHUNK)], pzb)

    def hashvec(i, _):
        o = 16 * i
        cxv = ((pxb[pl.ds(o, 16)] - 0.0) / 0.05).astype(jnp.int32)
        cyv = ((pyb[pl.ds(o, 16)] - (-40.0)) / 0.05).astype(jnp.int32)
        czv = ((pzb[pl.ds(o, 16)] - (-3.0)) / 0.1).astype(jnp.int32)
        inr = ((cxv >= 0) & (cxv < GX) & (cyv >= 0) & (cyv < GY)
               & (czv >= 0) & (czv < GZ))
        k = czv * GXY + cyv * GX + cxv
        keyb[pl.ds(o, 16)] = jnp.where(inr, k, SENT)
        return 0
    lax.fori_loop(0, CHUNK // 16, hashvec, 0)
    pltpu.sync_copy(keyb, keya.at[pl.ds(base, CHUNK)])

    @pl.when(w == 0)
    def _tail():
        s16[...] = iota * 0 + SENT
        pltpu.sync_copy(s16, keya.at[pl.ds(NP, 16)])

    plsc.subcore_barrier()

    @pl.when(w == 0)
    def _serial():
        def zero_hist(n):
            def zh(i, _):
                hist[pl.ds(16 * i, 16)] = iota * 0
                return 0
            lax.fori_loop(0, n // 16, zh, 0)

        def hist_pass(srck, dig):
            def blk(b, _):
                pltpu.sync_copy(srck.at[pl.ds(b * 1024, 1024)],
                                blkk.at[pl.ds(0, 1024)])
                def vec(j, _2):
                    kv = blkk[pl.ds(16 * j, 16)]
                    _ro, dsd, cnt, lastm = rank_dedup(dig(kv))
                    plsc.addupdate_scatter(hist, [dsd], cnt, mask=lastm)
                    return 0
                lax.fori_loop(0, 64, vec, 0)
                return 0
            lax.fori_loop(0, NBLK, blk, 0)

        def scan_hist(n):
            def sv(i, carry):
                v = hist[pl.ds(16 * i, 16)]
                c = plsc.cumsum(v)
                hist[pl.ds(16 * i, 16)] = c - v + carry
                return carry + jnp.max(c)
            lax.fori_loop(0, n // 16, sv, jnp.int32(0))

        def permute_pass(srck, srci, dstk, dsti, dig):
            def blk(b, _):
                pltpu.sync_copy(srck.at[pl.ds(b * 1024, 1024)],
                                blkk.at[pl.ds(0, 1024)])
                if srci is not None:
                    pltpu.sync_copy(srci.at[pl.ds(b * 1024, 1024)], blki)
                def sub(m, _2):
                    def vec(j, _3):
                        o = m * 128 + 16 * j
                        kv = blkk[pl.ds(o, 16)]
                        d = dig(kv)
                        ro, dsd, cnt, lastm = rank_dedup(d)
                        dest = plsc.load_gather(hist, [d]) + ro
                        plsc.addupdate_scatter(hist, [dsd], cnt, mask=lastm)
                        destr[pl.ds(16 * j, 16)] = dest
                        stg1[pl.ds(16 * j, 16)] = kv
                        if srci is None:
                            stg2[pl.ds(16 * j, 16)] = b * 1024 + o + iota
                        else:
                            stg2[pl.ds(16 * j, 16)] = blki[pl.ds(o, 16)]
                        return 0
                    lax.fori_loop(0, 8, vec, 0)
                    pltpu.sync_copy(stg1, dstk.at[destr])
                    pltpu.sync_copy(stg2, dsti.at[destr])
                    return 0
                lax.fori_loop(0, 8, sub, 0)
                return 0
            lax.fori_loop(0, NBLK, blk, 0)

        dig1 = lambda kv: kv & (K1 - 1)
        dig2 = lambda kv: kv >> SHIFT2

        zero_hist(K1)
        hist_pass(keya, dig1)
        scan_hist(K1)
        permute_pass(keya, None, key1, idx1, dig1)

        zero_hist(K2)
        hist_pass(key1, dig2)
        scan_hist(K2)
        permute_pass(key1, idx1, keya, idxb, dig2)

        def blk(b, carry):
            pltpu.sync_copy(keya.at[pl.ds(b * 1024, 1040)], blkk)
            pltpu.sync_copy(idxb.at[pl.ds(b * 1024, 1024)], blki)
            def sub(m, carry2):
                def vec(j, c3):
                    vidbase, gsc, prevl = c3
                    o = m * 128 + 16 * j
                    g0 = b * 1024 + o
                    kv = blkk[pl.ds(o, 16)]
                    nx = blkk[pl.ds(o + 1, 16)]
                    pv = jnp.where(iota == 0, prevl,
                                   take16(kv, jnp.maximum(iota - 1, 0)))
                    newg = kv != pv
                    s = plsc.cumsum(newg.astype(jnp.int32))
                    vid = vidbase + s - 1
                    gidx = g0 + iota
                    gs = jnp.maximum(plsc.cummax(jnp.where(newg, gidx, -1)),
                                     gsc)
                    pos = gidx - gs
                    validk = kv < SENT
                    vv = vid < MAXV
                    valid = validk & vv & (pos < MAXP)
                    dump = gidx & DMASK
                    destr[pl.ds(16 * j, 16)] = jnp.where(
                        valid, (vid * MAXP + pos) * 4, (VOXROWS + dump) * 4)
                    sidxr[pl.ds(16 * j, 16)] = blki[pl.ds(o, 16)]
                    lastm = kv != nx
                    stg2[pl.ds(16 * j, 16)] = jnp.where(
                        validk & vv & lastm, vid, MAXV + dump)
                    stg1[pl.ds(16 * j, 16)] = jnp.minimum(pos + 1, MAXP)
                    fm = newg & validk & vv
                    crdb[pl.ds(16 * j, 16)] = jnp.where(
                        fm, vid * 3, (MAXV + dump) * 3)
                    zv = kv // GXY
                    rem = kv - zv * GXY
                    yv = rem // GX
                    xv = rem - yv * GX
                    cz[pl.ds(16 * j, 16)] = zv
                    cy[pl.ds(16 * j, 16)] = yv
                    cx[pl.ds(16 * j, 16)] = xv
                    nvb = vidbase + jnp.max(s)
                    ngs = jnp.max(gs)
                    npl = jnp.max(jnp.where(iota == 15, kv,
                                            jnp.int32(-2147483647 - 1)))
                    return (nvb, ngs, npl)
                c3 = lax.fori_loop(0, 8, vec, carry2)
                for c in range(4):
                    def gadd(j, _4, c=c):
                        gidxb[pl.ds(16 * j, 16)] = (
                            sidxr[pl.ds(16 * j, 16)] + c * NP)
                        dstb[pl.ds(16 * j, 16)] = (
                            destr[pl.ds(16 * j, 16)] + c)
                        return 0
                    lax.fori_loop(0, 8, gadd, 0)
                    pltpu.sync_copy(pfl_hbm.at[gidxb], bcol)
                    pltpu.sync_copy(bcol, vox.at[dstb])
                pltpu.sync_copy(stg1, counts.at[stg2])
                for c, col in enumerate((cz, cy, cx)):
                    def cadd(j, _4, c=c):
                        dstb[pl.ds(16 * j, 16)] = (
                            crdb[pl.ds(16 * j, 16)] + c)
                        return 0
                    lax.fori_loop(0, 8, cadd, 0)
                    pltpu.sync_copy(col, coors.at[dstb])
                return c3
            return lax.fori_loop(0, 8, sub, carry)
        lax.fori_loop(0, NBLK, blk,
                      (jnp.int32(0), jnp.int32(-1), jnp.int32(-1)))


def kernel(points):
    pad = NP - N
    px = jnp.pad(points[:, 0], (0, pad), constant_values=-100.0)
    py = jnp.pad(points[:, 1], (0, pad), constant_values=-100.0)
    pz = jnp.pad(points[:, 2], (0, pad), constant_values=-100.0)
    pi = jnp.pad(points[:, 3], (0, pad), constant_values=0.0)
    pfl = jnp.concatenate([px, py, pz, pi])
    zf = jnp.zeros((4096,), jnp.float32)
    zi = jnp.zeros((4096,), jnp.int32)
    voxf, counts, coorsf = _voxelize_sc(pfl, zf, zi)
    voxels = voxf[:VOXROWS * 4].reshape(MAXV, MAXP, 4)
    coors = coorsf[:MAXV * 3].reshape(MAXV, 3)
    return voxels, coors, counts[:MAXV]
